# baseline (speedup 1.0000x reference)
import functools

import jax
import jax.numpy as jnp
from jax import lax
from jax.experimental import pallas as pl
from jax.experimental.pallas import tpu as pltpu
from jax.experimental.pallas import tpu_sc as plsc

N_NODES = 10000
IN_FEATS = 128
NUM_HEADS = 8
OUT_FEATS = 16
N_EDGES = 320000
NEG_SLOPE = 0.2

ROW = 144
CH = 80
NCH = N_EDGES // CH
NTILES = 32
NPT = NCH // NTILES
N_PAD = 10112
ROWS_PER_TILE = N_PAD // 16


def _proj_body(x_ref, wt_ref, alr_ref, featel_ref, er16_ref):
    f = jnp.dot(x_ref[...], wt_ref[...], preferred_element_type=jnp.float32)
    g = jnp.dot(f, alr_ref[...], preferred_element_type=jnp.float32)
    featel_ref[...] = jnp.concatenate([f, g[:, :16]], axis=1)
    er16_ref[...] = jnp.concatenate(
        [g[:, 8:16], jnp.zeros((f.shape[0], 8), jnp.float32)], axis=1)


def _finish_body(acc_ref, bias_ref, out_ref):
    u = acc_ref[0] + acc_ref[1]
    s = u[:, 128:136]
    r = jnp.where(s != 0.0, 1.0 / s, 0.0)
    parts = [u[:, h * 16:(h + 1) * 16] * r[:, h:h + 1] for h in range(NUM_HEADS)]
    out_ref[...] = jnp.concatenate(parts, axis=1) + bias_ref[...]


def _edge_body(featel, er16, idxc, out,
               ibuf0, ibuf1, srcv0, srcv1, dstv0, dstv1,
               gbuf0, gbuf1, ebuf0, ebuf1, acc,
               isem0, isem1, gsem0, gsem1, esem0, esem1, ssem0, ssem1):
    cid = lax.axis_index("c")
    sid = lax.axis_index("s")
    wid = sid * 2 + cid

    ibuf = (ibuf0, ibuf1)
    srcv = (srcv0, srcv1)
    dstv = (dstv0, dstv1)
    gbuf = (gbuf0, gbuf1)
    ebuf = (ebuf0, ebuf1)
    isem = (isem0, isem1)
    gsem = (gsem0, gsem1)
    esem = (esem0, esem1)
    ssem = (ssem0, ssem1)

    iota = lax.iota(jnp.int32, 16)
    mask8 = iota < 8
    zv = jnp.zeros((16,), jnp.float32)

    for i in range(CH):
        for j in range(ROW // 16):
            gbuf0[i, pl.ds(j * 16, 16)] = zv
    row0 = sid * ROWS_PER_TILE
    nzfull = (ROWS_PER_TILE // CH) * CH
    for r in range(0, nzfull, CH):
        pltpu.sync_copy(gbuf0, acc.at[pl.ds(row0 + r, CH)])
    pltpu.sync_copy(gbuf0.at[pl.ds(0, ROWS_PER_TILE - nzfull)],
                    acc.at[pl.ds(row0 + nzfull, ROWS_PER_TILE - nzfull)])
    plsc.subcore_barrier()

    def chunk_of(j):
        return wid * NPT + j

    def build_src(b):
        for i in range(CH // 16):
            srcv[b][pl.ds(i * 16, 16)] = ibuf[b][pl.ds(i * 16, 16)]

    def build_dst(b):
        for i in range(CH // 16):
            dstv[b][pl.ds(i * 16, 16)] = ibuf[b][pl.ds(CH + i * 16, 16)]

    def start_gathers(b):
        pltpu.async_copy(featel.at[srcv[b]], gbuf[b], gsem[b])
        pltpu.async_copy(er16.at[ibuf[b].at[pl.ds(CH, CH)]], ebuf[b], esem[b])

    def wait_gathers(b):
        pltpu.make_async_copy(featel.at[srcv[b]], gbuf[b], gsem[b]).wait()
        pltpu.make_async_copy(er16.at[ibuf[b].at[pl.ds(CH, CH)]], ebuf[b],
                              esem[b]).wait()

    def wait_scatter(b):
        pltpu.make_async_copy(gbuf[b], acc.at[dstv[b]], ssem[b]).wait()

    def start_scatter(b):
        pltpu.async_copy(gbuf[b], acc.at[dstv[b]], ssem[b], add=True)

    def compute_chunk(b):
        def edge_iter(e, inner):
            el16 = gbuf[b][e, pl.ds(128, 16)]
            er16v = ebuf[b][e, pl.ds(0, 16)]
            z = el16 + er16v
            w = jnp.exp(jnp.maximum(z, z * NEG_SLOPE))
            for h in range(NUM_HEADS):
                off = h * 16
                gbuf[b][e, pl.ds(off, 16)] = gbuf[b][e, pl.ds(off, 16)] * w[h]
            gbuf[b][e, pl.ds(128, 16)] = jnp.where(mask8, w, 0.0)
            return inner
        lax.fori_loop(0, CH, edge_iter, 0)

    pltpu.sync_copy(idxc.at[chunk_of(0)], ibuf[0])
    pltpu.async_copy(idxc.at[chunk_of(1)], ibuf[1], isem[1])
    build_src(0)
    start_gathers(0)

    def pair_body(p, carry):
        for b in range(2):
            j = 2 * p + b
            wait_gathers(b)
            build_dst(b)

            if b == 0:
                pltpu.async_copy(idxc.at[chunk_of(j + 2)], ibuf[b], isem[b])
            else:
                @pl.when(p < NPT // 2 - 1)
                def _():
                    pltpu.async_copy(idxc.at[chunk_of(j + 2)], ibuf[b],
                                     isem[b])

            pltpu.make_async_copy(idxc.at[chunk_of(j + 1)], ibuf[1 - b],
                                  isem[1 - b]).wait()
            build_src(1 - b)
            if b == 0:
                @pl.when(p >= 1)
                def _():
                    wait_scatter(1 - b)
            else:
                wait_scatter(1 - b)
            start_gathers(1 - b)

            compute_chunk(b)
            start_scatter(b)
        return carry

    lax.fori_loop(0, NPT // 2, pair_body, 0)

    wait_gathers(0)
    build_dst(0)
    compute_chunk(0)
    start_scatter(0)

    wait_scatter(1)
    wait_scatter(0)
    plsc.subcore_barrier()

    pltpu.sync_copy(acc.at[pl.ds(row0, ROWS_PER_TILE)],
                    out.at[cid, pl.ds(row0, ROWS_PER_TILE)])


def kernel(x, edge_index, W, attn_l, attn_r, bias):
    src = edge_index[0].astype(jnp.int32)
    dst = edge_index[1].astype(jnp.int32)

    eye = jnp.eye(NUM_HEADS, dtype=jnp.float32)
    al = (eye[:, None, :] * attn_l[0][:, :, None]).reshape(IN_FEATS, NUM_HEADS)
    ar = (eye[:, None, :] * attn_r[0][:, :, None]).reshape(IN_FEATS, NUM_HEADS)
    alr = jnp.zeros((IN_FEATS, IN_FEATS), jnp.float32)
    alr = alr.at[:, :NUM_HEADS].set(al).at[:, NUM_HEADS:2 * NUM_HEADS].set(ar)

    blk = 1000
    grid = N_NODES // blk
    featel, er16 = pl.pallas_call(
        _proj_body,
        grid=(grid,),
        in_specs=[
            pl.BlockSpec((blk, IN_FEATS), lambda i: (i, 0)),
            pl.BlockSpec((IN_FEATS, IN_FEATS), lambda i: (0, 0)),
            pl.BlockSpec((IN_FEATS, IN_FEATS), lambda i: (0, 0)),
        ],
        out_specs=[
            pl.BlockSpec((blk, ROW), lambda i: (i, 0)),
            pl.BlockSpec((blk, 16), lambda i: (i, 0)),
        ],
        out_shape=[
            jax.ShapeDtypeStruct((N_NODES, ROW), jnp.float32),
            jax.ShapeDtypeStruct((N_NODES, 16), jnp.float32),
        ],
    )(x, W.T, alr)

    idxc = jnp.concatenate(
        [src.reshape(NCH, CH), dst.reshape(NCH, CH)], axis=1)

    edge_kernel = functools.partial(
        pl.kernel,
        out_type=jax.ShapeDtypeStruct((2, N_PAD, ROW), jnp.float32),
        mesh=plsc.VectorSubcoreMesh(core_axis_name="c", subcore_axis_name="s"),
        compiler_params=pltpu.CompilerParams(use_tc_tiling_on_sc=False),
        scratch_types=[
            pltpu.VMEM((2 * CH,), jnp.int32),
            pltpu.VMEM((2 * CH,), jnp.int32),
            pltpu.VMEM((CH,), jnp.int32),
            pltpu.VMEM((CH,), jnp.int32),
            pltpu.VMEM((CH,), jnp.int32),
            pltpu.VMEM((CH,), jnp.int32),
            pltpu.VMEM((CH, ROW), jnp.float32),
            pltpu.VMEM((CH, ROW), jnp.float32),
            pltpu.VMEM((CH, 16), jnp.float32),
            pltpu.VMEM((CH, 16), jnp.float32),
            pltpu.VMEM_SHARED((N_PAD, ROW), jnp.float32),
        ] + [pltpu.SemaphoreType.DMA] * 8,
    )(_edge_body)
    acc2 = edge_kernel(featel, er16, idxc)

    out = pl.pallas_call(
        _finish_body,
        grid=(grid,),
        in_specs=[
            pl.BlockSpec((2, blk, ROW), lambda i: (0, i, 0)),
            pl.BlockSpec((1, IN_FEATS), lambda i: (0, 0)),
        ],
        out_specs=pl.BlockSpec((blk, IN_FEATS), lambda i: (i, 0)),
        out_shape=jax.ShapeDtypeStruct((N_NODES, IN_FEATS), jnp.float32),
    )(acc2, bias.reshape(1, IN_FEATS))

    return out.reshape(N_NODES, NUM_HEADS, OUT_FEATS)


# speedup vs baseline: 1.0692x; 1.0692x over previous
import functools

import jax
import jax.numpy as jnp
from jax import lax
from jax.experimental import pallas as pl
from jax.experimental.pallas import tpu as pltpu
from jax.experimental.pallas import tpu_sc as plsc

N_NODES = 10000
IN_FEATS = 128
NUM_HEADS = 8
OUT_FEATS = 16
N_EDGES = 320000
NEG_SLOPE = 0.2

ROW = 144
CH = 64
NCH = N_EDGES // CH
NTILES = 32
NFULL = NCH // NTILES
NEXTRA = NCH % NTILES
N_PAD = 10112
ROWS_PER_TILE = N_PAD // 16


def _proj_body(x_ref, wt_ref, al_ref, ar_ref, featel_ref, er16_ref):
    f = jnp.dot(x_ref[...], wt_ref[...], preferred_element_type=jnp.float32)
    els, ers = [], []
    for h in range(NUM_HEADS):
        fh = f[:, h * 16:(h + 1) * 16]
        els.append(jnp.sum(fh * al_ref[h:h + 1, :], axis=1, keepdims=True))
        ers.append(jnp.sum(fh * ar_ref[h:h + 1, :], axis=1, keepdims=True))
    el = jnp.concatenate(els, axis=1)
    er = jnp.concatenate(ers, axis=1)
    featel_ref[...] = jnp.concatenate([f, el, er], axis=1)
    er16_ref[...] = jnp.concatenate([er, jnp.zeros_like(er)], axis=1)


def _finish_body(acc_ref, bias_ref, out_ref):
    u = acc_ref[0] + acc_ref[1]
    s = u[:, 128:136]
    r = jnp.where(s != 0.0, 1.0 / s, 0.0)
    parts = [u[:, h * 16:(h + 1) * 16] * r[:, h:h + 1] for h in range(NUM_HEADS)]
    out_ref[...] = jnp.concatenate(parts, axis=1) + bias_ref[...]


def _edge_body(featel, er16, srcs, dsts, out,
               srcv0, srcv1, srcv2, srcv3, dstv0, dstv1, dstv2, dstv3,
               gbuf0, gbuf1, ebuf0, ebuf1, obuf0, obuf1, acc,
               xsem0, xsem1, xsem2, xsem3,
               gsem0, gsem1, esem0, esem1, ssem0, ssem1):
    cid = lax.axis_index("c")
    sid = lax.axis_index("s")
    wid = sid * 2 + cid

    srcv = (srcv0, srcv1, srcv2, srcv3)
    dstv = (dstv0, dstv1, dstv2, dstv3)
    xsem = (xsem0, xsem1, xsem2, xsem3)
    gbuf = (gbuf0, gbuf1)
    ebuf = (ebuf0, ebuf1)
    obuf = (obuf0, obuf1)
    gsem = (gsem0, gsem1)
    esem = (esem0, esem1)
    ssem = (ssem0, ssem1)

    iota = lax.iota(jnp.int32, 16)
    mask8 = iota < 8
    zv = jnp.zeros((16,), jnp.float32)
    has_extra = wid < NEXTRA

    for i in range(CH):
        for j in range(ROW // 16):
            obuf0[i, pl.ds(j * 16, 16)] = zv
    row0 = sid * ROWS_PER_TILE
    nzfull = (ROWS_PER_TILE // CH) * CH
    for r in range(0, nzfull, CH):
        pltpu.sync_copy(obuf0, acc.at[pl.ds(row0 + r, CH)])
    pltpu.sync_copy(obuf0.at[pl.ds(0, ROWS_PER_TILE - nzfull)],
                    acc.at[pl.ds(row0 + nzfull, ROWS_PER_TILE - nzfull)])
    plsc.subcore_barrier()

    def base_of(j):
        return (wid + j * NTILES) * CH

    def start_idx(j, s):
        pltpu.async_copy(srcs.at[pl.ds(base_of(j), CH)], srcv[s], xsem[s])
        pltpu.async_copy(dsts.at[pl.ds(base_of(j), CH)], dstv[s], xsem[s])

    def wait_idx(j, s):
        pltpu.make_async_copy(srcs.at[pl.ds(base_of(j), CH)], srcv[s],
                              xsem[s]).wait()
        pltpu.make_async_copy(dsts.at[pl.ds(base_of(j), CH)], dstv[s],
                              xsem[s]).wait()

    def start_gathers(b, s):
        pltpu.async_copy(featel.at[srcv[s]], gbuf[b], gsem[b])
        pltpu.async_copy(er16.at[dstv[s]], ebuf[b], esem[b])

    def wait_gathers(b, s):
        pltpu.make_async_copy(featel.at[srcv[s]], gbuf[b], gsem[b]).wait()
        pltpu.make_async_copy(er16.at[dstv[s]], ebuf[b], esem[b]).wait()

    def start_scatter(b, s):
        pltpu.async_copy(obuf[b], acc.at[dstv[s]], ssem[b], add=True)

    def wait_scatter(b, s):
        pltpu.make_async_copy(obuf[b], acc.at[dstv[s]], ssem[b]).wait()

    def compute_chunk(b):
        def edge_iter(e, inner):
            el16 = gbuf[b][e, pl.ds(128, 16)]
            er16v = ebuf[b][e, pl.ds(0, 16)]
            z = el16 + er16v
            w = jnp.exp(jnp.maximum(z, z * NEG_SLOPE))
            obuf[b][e, pl.ds(128, 16)] = jnp.where(mask8, w, 0.0)
            for h in range(NUM_HEADS):
                off = h * 16
                obuf[b][e, pl.ds(off, 16)] = gbuf[b][e, pl.ds(off, 16)] * w[h]
            return inner
        lax.fori_loop(0, CH, edge_iter, 0)

    pltpu.sync_copy(srcs.at[pl.ds(base_of(0), CH)], srcv[0])
    pltpu.sync_copy(dsts.at[pl.ds(base_of(0), CH)], dstv[0])
    start_idx(1, 1)
    start_gathers(0, 0)

    NQ = NFULL // 4

    def quad_body(q, carry):
        for t in range(4):
            j = 4 * q + t
            b = t & 1
            wait_gathers(b, t)

            if t < 2:
                @pl.when(q >= 1)
                def _():
                    wait_scatter(b, (t + 2) % 4)
            else:
                wait_scatter(b, (t + 2) % 4)

            if t < 2:
                start_idx(j + 2, (t + 2) % 4)
            else:
                nok = ((q < NQ - 1) | has_extra) if t == 2 else (q < NQ - 1)

                @pl.when(nok)
                def _():
                    start_idx(j + 2, (t + 2) % 4)

            if t < 3:
                wait_idx(j + 1, (t + 1) % 4)
                start_gathers(1 - b, (t + 1) % 4)
            else:
                gok = (q < NQ - 1) | has_extra

                @pl.when(gok)
                def _():
                    wait_idx(j + 1, (t + 1) % 4)
                    start_gathers(1 - b, (t + 1) % 4)

            compute_chunk(b)
            start_scatter(b, t)
        return carry

    lax.fori_loop(0, NQ, quad_body, 0)

    @pl.when(has_extra)
    def _():
        wait_gathers(0, 0)
        wait_scatter(0, 2)
        compute_chunk(0)
        start_scatter(0, 0)

    wait_scatter(1, 3)
    @pl.when(has_extra)
    def _():
        wait_scatter(0, 0)

    @pl.when(jnp.logical_not(has_extra))
    def _():
        wait_scatter(0, 2)
    plsc.subcore_barrier()

    pltpu.sync_copy(acc.at[pl.ds(row0, ROWS_PER_TILE)],
                    out.at[cid, pl.ds(row0, ROWS_PER_TILE)])


def kernel(x, edge_index, W, attn_l, attn_r, bias):
    src = edge_index[0].astype(jnp.int32)
    dst = edge_index[1].astype(jnp.int32)

    blk = 1000
    grid = N_NODES // blk
    featel, er16 = pl.pallas_call(
        _proj_body,
        grid=(grid,),
        in_specs=[
            pl.BlockSpec((blk, IN_FEATS), lambda i: (i, 0)),
            pl.BlockSpec((IN_FEATS, IN_FEATS), lambda i: (0, 0)),
            pl.BlockSpec((NUM_HEADS, OUT_FEATS), lambda i: (0, 0)),
            pl.BlockSpec((NUM_HEADS, OUT_FEATS), lambda i: (0, 0)),
        ],
        out_specs=[
            pl.BlockSpec((blk, ROW), lambda i: (i, 0)),
            pl.BlockSpec((blk, 16), lambda i: (i, 0)),
        ],
        out_shape=[
            jax.ShapeDtypeStruct((N_NODES, ROW), jnp.float32),
            jax.ShapeDtypeStruct((N_NODES, 16), jnp.float32),
        ],
    )(x, W.T, attn_l.reshape(NUM_HEADS, OUT_FEATS),
      attn_r.reshape(NUM_HEADS, OUT_FEATS))

    edge_kernel = functools.partial(
        pl.kernel,
        out_type=jax.ShapeDtypeStruct((2, N_PAD, ROW), jnp.float32),
        mesh=plsc.VectorSubcoreMesh(core_axis_name="c", subcore_axis_name="s"),
        compiler_params=pltpu.CompilerParams(use_tc_tiling_on_sc=False),
        scratch_types=[
            pltpu.VMEM((CH,), jnp.int32),
            pltpu.VMEM((CH,), jnp.int32),
            pltpu.VMEM((CH,), jnp.int32),
            pltpu.VMEM((CH,), jnp.int32),
            pltpu.VMEM((CH,), jnp.int32),
            pltpu.VMEM((CH,), jnp.int32),
            pltpu.VMEM((CH,), jnp.int32),
            pltpu.VMEM((CH,), jnp.int32),
            pltpu.VMEM((CH, ROW), jnp.float32),
            pltpu.VMEM((CH, ROW), jnp.float32),
            pltpu.VMEM((CH, 16), jnp.float32),
            pltpu.VMEM((CH, 16), jnp.float32),
            pltpu.VMEM((CH, ROW), jnp.float32),
            pltpu.VMEM((CH, ROW), jnp.float32),
            pltpu.VMEM_SHARED((N_PAD, ROW), jnp.float32),
        ] + [pltpu.SemaphoreType.DMA] * 10,
    )(_edge_body)
    acc2 = edge_kernel(featel, er16, src, dst)

    out = pl.pallas_call(
        _finish_body,
        grid=(grid,),
        in_specs=[
            pl.BlockSpec((2, blk, ROW), lambda i: (0, i, 0)),
            pl.BlockSpec((1, IN_FEATS), lambda i: (0, 0)),
        ],
        out_specs=pl.BlockSpec((blk, IN_FEATS), lambda i: (i, 0)),
        out_shape=jax.ShapeDtypeStruct((N_NODES, IN_FEATS), jnp.float32),
    )(acc2, bias.reshape(1, IN_FEATS))

    return out.reshape(N_NODES, NUM_HEADS, OUT_FEATS)


# speedup vs baseline: 1.2124x; 1.1339x over previous
import functools

import jax
import jax.numpy as jnp
from jax import lax
from jax.experimental import pallas as pl
from jax.experimental.pallas import tpu as pltpu
from jax.experimental.pallas import tpu_sc as plsc

N_NODES = 10000
IN_FEATS = 128
NUM_HEADS = 8
OUT_FEATS = 16
N_EDGES = 320000
NEG_SLOPE = 0.2

ROW = 144
CH = 64
NCH = N_EDGES // CH
NTILES = 32
NFULL = NCH // NTILES
NEXTRA = NCH % NTILES
N_PAD = 10112
ROWS_PER_TILE = N_PAD // 16


def _proj_body(x_ref, wt_ref, al_ref, ar_ref, featel_ref, er16_ref):
    f = jnp.dot(x_ref[...], wt_ref[...], preferred_element_type=jnp.float32)
    row = lax.broadcasted_iota(jnp.int32, (IN_FEATS, 16), 0)
    col = lax.broadcasted_iota(jnp.int32, (IN_FEATS, 16), 1)
    head = row // OUT_FEATS
    alr = jnp.where(col < NUM_HEADS,
                    jnp.where(head == col, al_ref[...], 0.0),
                    jnp.where(head == col - NUM_HEADS, ar_ref[...], 0.0))
    g = jnp.dot(f, alr, preferred_element_type=jnp.float32)
    featel_ref[...] = jnp.concatenate([f, g], axis=1)
    er16_ref[...] = jnp.concatenate(
        [g[:, 8:16], jnp.zeros((f.shape[0], 8), jnp.float32)], axis=1)


def _finish_body(acc_ref, bias_ref, out_ref):
    u = acc_ref[0] + acc_ref[1]
    s = u[:, 128:136]
    r = jnp.where(s != 0.0, 1.0 / s, 0.0)
    parts = [u[:, h * 16:(h + 1) * 16] * r[:, h:h + 1] for h in range(NUM_HEADS)]
    out_ref[...] = jnp.concatenate(parts, axis=1) + bias_ref[...]


def _edge_body(featel, er16, srcs, dsts, out,
               srcv0, srcv1, srcv2, srcv3, dstv0, dstv1, dstv2, dstv3,
               gbuf0, gbuf1, ebuf0, ebuf1, obuf0, obuf1, acc,
               xsem0, xsem1, xsem2, xsem3,
               gsem0, gsem1, esem0, esem1, ssem0, ssem1):
    cid = lax.axis_index("c")
    sid = lax.axis_index("s")
    wid = sid * 2 + cid

    srcv = (srcv0, srcv1, srcv2, srcv3)
    dstv = (dstv0, dstv1, dstv2, dstv3)
    xsem = (xsem0, xsem1, xsem2, xsem3)
    gbuf = (gbuf0, gbuf1)
    ebuf = (ebuf0, ebuf1)
    obuf = (obuf0, obuf1)
    gsem = (gsem0, gsem1)
    esem = (esem0, esem1)
    ssem = (ssem0, ssem1)

    iota = lax.iota(jnp.int32, 16)
    mask8 = iota < 8
    zv = jnp.zeros((16,), jnp.float32)
    has_extra = wid < NEXTRA

    for i in range(CH):
        for j in range(ROW // 16):
            obuf0[i, pl.ds(j * 16, 16)] = zv
    row0 = sid * ROWS_PER_TILE
    nzfull = (ROWS_PER_TILE // CH) * CH
    for r in range(0, nzfull, CH):
        pltpu.sync_copy(obuf0, acc.at[pl.ds(row0 + r, CH)])
    pltpu.sync_copy(obuf0.at[pl.ds(0, ROWS_PER_TILE - nzfull)],
                    acc.at[pl.ds(row0 + nzfull, ROWS_PER_TILE - nzfull)])
    plsc.subcore_barrier()

    def base_of(j):
        return (wid + j * NTILES) * CH

    def start_idx(j, s):
        pltpu.async_copy(srcs.at[pl.ds(base_of(j), CH)], srcv[s], xsem[s])
        pltpu.async_copy(dsts.at[pl.ds(base_of(j), CH)], dstv[s], xsem[s])

    def wait_idx(j, s):
        pltpu.make_async_copy(srcs.at[pl.ds(base_of(j), CH)], srcv[s],
                              xsem[s]).wait()
        pltpu.make_async_copy(dsts.at[pl.ds(base_of(j), CH)], dstv[s],
                              xsem[s]).wait()

    def start_gathers(b, s):
        pltpu.async_copy(featel.at[srcv[s]], gbuf[b], gsem[b])
        pltpu.async_copy(er16.at[dstv[s]], ebuf[b], esem[b])

    def wait_gathers(b, s):
        pltpu.make_async_copy(featel.at[srcv[s]], gbuf[b], gsem[b]).wait()
        pltpu.make_async_copy(er16.at[dstv[s]], ebuf[b], esem[b]).wait()

    def start_scatter(b, s):
        pltpu.async_copy(obuf[b], acc.at[dstv[s]], ssem[b], add=True)

    def wait_scatter(b, s):
        pltpu.make_async_copy(obuf[b], acc.at[dstv[s]], ssem[b]).wait()

    def compute_chunk(b):
        def edge_iter(e, inner):
            el16 = gbuf[b][e, pl.ds(128, 16)]
            er16v = ebuf[b][e, pl.ds(0, 16)]
            z = el16 + er16v
            w = jnp.exp(jnp.maximum(z, z * NEG_SLOPE))
            obuf[b][e, pl.ds(128, 16)] = jnp.where(mask8, w, 0.0)
            for h in range(NUM_HEADS):
                off = h * 16
                obuf[b][e, pl.ds(off, 16)] = gbuf[b][e, pl.ds(off, 16)] * w[h]
            return inner
        lax.fori_loop(0, CH, edge_iter, 0)

    pltpu.sync_copy(srcs.at[pl.ds(base_of(0), CH)], srcv[0])
    pltpu.sync_copy(dsts.at[pl.ds(base_of(0), CH)], dstv[0])
    start_idx(1, 1)
    start_gathers(0, 0)

    NQ = NFULL // 4

    def quad_body(q, carry):
        for t in range(4):
            j = 4 * q + t
            b = t & 1
            wait_gathers(b, t)

            if t < 2:
                @pl.when(q >= 1)
                def _():
                    wait_scatter(b, (t + 2) % 4)
            else:
                wait_scatter(b, (t + 2) % 4)

            if t < 2:
                start_idx(j + 2, (t + 2) % 4)
            else:
                nok = ((q < NQ - 1) | has_extra) if t == 2 else (q < NQ - 1)

                @pl.when(nok)
                def _():
                    start_idx(j + 2, (t + 2) % 4)

            if t < 3:
                wait_idx(j + 1, (t + 1) % 4)
                start_gathers(1 - b, (t + 1) % 4)
            else:
                gok = (q < NQ - 1) | has_extra

                @pl.when(gok)
                def _():
                    wait_idx(j + 1, (t + 1) % 4)
                    start_gathers(1 - b, (t + 1) % 4)

            compute_chunk(b)
            start_scatter(b, t)
        return carry

    lax.fori_loop(0, NQ, quad_body, 0)

    @pl.when(has_extra)
    def _():
        wait_gathers(0, 0)
        wait_scatter(0, 2)
        compute_chunk(0)
        start_scatter(0, 0)

    wait_scatter(1, 3)
    @pl.when(has_extra)
    def _():
        wait_scatter(0, 0)

    @pl.when(jnp.logical_not(has_extra))
    def _():
        wait_scatter(0, 2)
    plsc.subcore_barrier()

    pltpu.sync_copy(acc.at[pl.ds(row0, ROWS_PER_TILE)],
                    out.at[cid, pl.ds(row0, ROWS_PER_TILE)])


def kernel(x, edge_index, W, attn_l, attn_r, bias):
    src = edge_index[0].astype(jnp.int32)
    dst = edge_index[1].astype(jnp.int32)

    blk = 1000
    grid = N_NODES // blk
    featel, er16 = pl.pallas_call(
        _proj_body,
        grid=(grid,),
        in_specs=[
            pl.BlockSpec((blk, IN_FEATS), lambda i: (i, 0)),
            pl.BlockSpec((IN_FEATS, IN_FEATS), lambda i: (0, 0)),
            pl.BlockSpec((IN_FEATS, 16), lambda i: (0, 0)),
            pl.BlockSpec((IN_FEATS, 16), lambda i: (0, 0)),
        ],
        out_specs=[
            pl.BlockSpec((blk, ROW), lambda i: (i, 0)),
            pl.BlockSpec((blk, 16), lambda i: (i, 0)),
        ],
        out_shape=[
            jax.ShapeDtypeStruct((N_NODES, ROW), jnp.float32),
            jax.ShapeDtypeStruct((N_NODES, 16), jnp.float32),
        ],
    )(x, W.T,
      jnp.broadcast_to(attn_l.reshape(IN_FEATS, 1), (IN_FEATS, 16)),
      jnp.broadcast_to(attn_r.reshape(IN_FEATS, 1), (IN_FEATS, 16)))

    edge_kernel = functools.partial(
        pl.kernel,
        out_type=jax.ShapeDtypeStruct((2, N_PAD, ROW), jnp.float32),
        mesh=plsc.VectorSubcoreMesh(core_axis_name="c", subcore_axis_name="s"),
        compiler_params=pltpu.CompilerParams(use_tc_tiling_on_sc=False),
        scratch_types=[
            pltpu.VMEM((CH,), jnp.int32),
            pltpu.VMEM((CH,), jnp.int32),
            pltpu.VMEM((CH,), jnp.int32),
            pltpu.VMEM((CH,), jnp.int32),
            pltpu.VMEM((CH,), jnp.int32),
            pltpu.VMEM((CH,), jnp.int32),
            pltpu.VMEM((CH,), jnp.int32),
            pltpu.VMEM((CH,), jnp.int32),
            pltpu.VMEM((CH, ROW), jnp.float32),
            pltpu.VMEM((CH, ROW), jnp.float32),
            pltpu.VMEM((CH, 16), jnp.float32),
            pltpu.VMEM((CH, 16), jnp.float32),
            pltpu.VMEM((CH, ROW), jnp.float32),
            pltpu.VMEM((CH, ROW), jnp.float32),
            pltpu.VMEM_SHARED((N_PAD, ROW), jnp.float32),
        ] + [pltpu.SemaphoreType.DMA] * 10,
    )(_edge_body)
    acc2 = edge_kernel(featel, er16, src, dst)

    out = pl.pallas_call(
        _finish_body,
        grid=(grid,),
        in_specs=[
            pl.BlockSpec((2, blk, ROW), lambda i: (0, i, 0)),
            pl.BlockSpec((1, IN_FEATS), lambda i: (0, 0)),
        ],
        out_specs=pl.BlockSpec((blk, IN_FEATS), lambda i: (i, 0)),
        out_shape=jax.ShapeDtypeStruct((N_NODES, IN_FEATS), jnp.float32),
    )(acc2, bias.reshape(1, IN_FEATS))

    return out.reshape(N_NODES, NUM_HEADS, OUT_FEATS)
